# baseline (speedup 1.0000x reference)
import functools

import jax
import jax.numpy as jnp
from jax import lax
from jax.experimental import pallas as pl
from jax.experimental.pallas import tpu as pltpu

TOPK = 9


def _atss_body(tref, pref, posx_o, posy_o, posw_o, posh_o, giou_o, mask_o,
               *, npad: int, tb: int):
    tx = tref[:, 0:1]
    ty = tref[:, 1:2]
    tw = tref[:, 2:3]
    th = tref[:, 3:4]
    px = pref[0:1, :]
    py = pref[1:2, :]
    pw = pref[2:3, :]
    ph = pref[3:4, :]

    dx = tx - px
    dy = ty - py
    d = jnp.sqrt(dx * dx + dy * dy)

    iota = lax.broadcasted_iota(jnp.int32, (tb, npad), 1)

    cxs, cys, cws, chs = [], [], [], []
    for _ in range(TOPK):
        m = jnp.min(d, axis=1, keepdims=True)
        idx = jnp.min(jnp.where(d == m, iota, npad), axis=1, keepdims=True)
        eq = iota == idx
        cxs.append(jnp.sum(jnp.where(eq, px, 0.0), axis=1, keepdims=True))
        cys.append(jnp.sum(jnp.where(eq, py, 0.0), axis=1, keepdims=True))
        cws.append(jnp.sum(jnp.where(eq, pw, 0.0), axis=1, keepdims=True))
        chs.append(jnp.sum(jnp.where(eq, ph, 0.0), axis=1, keepdims=True))
        d = jnp.where(eq, jnp.inf, d)

    cx = jnp.concatenate(cxs, axis=1)
    cy = jnp.concatenate(cys, axis=1)
    cw = jnp.concatenate(cws, axis=1)
    ch = jnp.concatenate(chs, axis=1)

    b1_x1 = tx - tw / 2
    b1_x2 = tx + tw / 2
    b1_y1 = ty - th / 2
    b1_y2 = ty + th / 2
    b2_x1 = cx - cw / 2
    b2_x2 = cx + cw / 2
    b2_y1 = cy - ch / 2
    b2_y2 = cy + ch / 2
    inter = jnp.clip(jnp.minimum(b1_x2, b2_x2) - jnp.maximum(b1_x1, b2_x1), 0.0, None) * \
            jnp.clip(jnp.minimum(b1_y2, b2_y2) - jnp.maximum(b1_y1, b2_y1), 0.0, None)
    w1 = b1_x2 - b1_x1
    h1 = b1_y2 - b1_y1
    w2 = b2_x2 - b2_x1
    h2 = b2_y2 - b2_y1
    union = w1 * h1 + 1e-16 + w2 * h2 - inter
    iou = inter / union
    cw_e = jnp.maximum(b1_x2, b2_x2) - jnp.minimum(b1_x1, b2_x1)
    ch_e = jnp.maximum(b1_y2, b2_y2) - jnp.minimum(b1_y1, b2_y1)
    c_area = cw_e * ch_e + 1e-16
    giou = iou - (c_area - union) / c_area

    mu = jnp.sum(giou, axis=1, keepdims=True) / TOPK
    cdev = giou - mu
    var = jnp.sum(cdev * cdev, axis=1, keepdims=True) / (TOPK - 1)
    thr = mu + var
    maskb = giou > thr

    posx_o[...] = jnp.where(maskb, cx, 0.0)
    posy_o[...] = jnp.where(maskb, cy, 0.0)
    posw_o[...] = jnp.where(maskb, cw, 0.0)
    posh_o[...] = jnp.where(maskb, ch, 0.0)
    giou_o[...] = giou
    mask_o[...] = maskb.astype(jnp.int32)


@functools.partial(jax.jit, static_argnames=("interpret",))
def kernel(p_boxes, target, interpret=False):
    n = p_boxes.shape[0]
    nt = target.shape[0]
    npad = ((n + 127) // 128) * 128
    tb = 64

    pc = p_boxes[:, 2:6]
    px = jnp.pad(pc[:, 0], (0, npad - n), constant_values=1e9)
    py = jnp.pad(pc[:, 1], (0, npad - n), constant_values=1e9)
    pw = jnp.pad(pc[:, 2], (0, npad - n), constant_values=0.0)
    ph = jnp.pad(pc[:, 3], (0, npad - n), constant_values=0.0)
    pref = jnp.stack([px, py, pw, ph], axis=0)
    tref = target[:, 2:6]

    grid = (nt // tb,)
    out_shapes = [
        jax.ShapeDtypeStruct((nt, TOPK), jnp.float32),
        jax.ShapeDtypeStruct((nt, TOPK), jnp.float32),
        jax.ShapeDtypeStruct((nt, TOPK), jnp.float32),
        jax.ShapeDtypeStruct((nt, TOPK), jnp.float32),
        jax.ShapeDtypeStruct((nt, TOPK), jnp.float32),
        jax.ShapeDtypeStruct((nt, TOPK), jnp.int32),
    ]
    row_spec = pl.BlockSpec((tb, TOPK), lambda i: (i, 0))
    posx, posy, posw, posh, giou, maski = pl.pallas_call(
        functools.partial(_atss_body, npad=npad, tb=tb),
        grid=grid,
        in_specs=[
            pl.BlockSpec((tb, 4), lambda i: (i, 0)),
            pl.BlockSpec((4, npad), lambda i: (0, 0)),
        ],
        out_specs=[row_spec] * 6,
        out_shape=out_shapes,
        compiler_params=pltpu.CompilerParams(
            dimension_semantics=("arbitrary",)),
        interpret=interpret,
    )(tref, pref)

    pos = jnp.stack([posx, posy, posw, posh], axis=-1)
    return pos, giou, maski.astype(bool)


# speedup vs baseline: 2.3196x; 2.3196x over previous
import functools

import jax
import jax.numpy as jnp
import numpy as np
from jax import lax
from jax.experimental import pallas as pl
from jax.experimental.pallas import tpu as pltpu
from jax.experimental.pallas import tpu_sc as plsc

TOPK = 9


def _topk_body(tref, pref, idx_o, *, npad: int, tb: int):
    tx = tref[:, 0:1]
    ty = tref[:, 1:2]
    px = pref[0:1, :]
    py = pref[1:2, :]
    dx = tx - px
    dy = ty - py
    d = jnp.sqrt(dx * dx + dy * dy)
    iota = lax.broadcasted_iota(jnp.int32, (tb, npad), 1)
    cols = []
    for _ in range(TOPK):
        m = jnp.min(d, axis=1, keepdims=True)
        idxv = jnp.min(jnp.where(d == m, iota, npad), axis=1, keepdims=True)
        eq = iota == idxv
        d = jnp.where(eq, jnp.inf, d)
        cols.append(idxv)
    idx_o[...] = jnp.concatenate(cols, axis=1)


def _sc_epilogue(nt: int, npad: int):
    info = plsc.get_sparse_core_info()
    nc, ns = info.num_cores, info.num_subcores
    nw = nc * ns
    rows_w = nt // nw
    gidx = rows_w * TOPK
    mesh = plsc.VectorSubcoreMesh(core_axis_name="c", subcore_axis_name="s")

    @functools.partial(
        pl.kernel, mesh=mesh,
        out_type=jax.ShapeDtypeStruct((nt, 128), jnp.float32),
        scratch_types=[
            pltpu.VMEM((gidx,), jnp.int32),
            pltpu.VMEM((gidx, 128), jnp.float32),
            pltpu.VMEM((rows_w, 128), jnp.float32),
            pltpu.VMEM((rows_w, 128), jnp.float32),
            pltpu.SemaphoreType.DMA,
        ],
    )
    def sc_fn(idx_hbm, boxes_hbm, tgt_hbm, out_hbm,
              idx_v, rows_v, tv, ov, sem):
        c = lax.axis_index("c")
        s = lax.axis_index("s")
        wid = s * nc + c
        base = wid * rows_w
        pltpu.sync_copy(idx_hbm.at[pl.ds(wid * gidx, gidx)], idx_v)
        pltpu.async_copy(boxes_hbm.at[idx_v], rows_v, sem).wait()
        pltpu.sync_copy(tgt_hbm.at[pl.ds(base, rows_w)], tv)

        lane = lax.iota(jnp.int32, 16)
        valid = lane < TOPK
        comp = [((lane * 0) + c).reshape(16, 1) for c in range(4)]
        dnums = lax.GatherDimensionNumbers(
            offset_dims=(), collapsed_slice_dims=(0,), start_index_map=(0,))

        def permute(vec, idx):
            return lax.gather(vec, idx, dnums, (1,),
                              mode=lax.GatherScatterMode.PROMISE_IN_BOUNDS)

        def splat(vec, c):
            return permute(vec, comp[c])

        perms = [((lane + sh) % 16).reshape(16, 1) for sh in (8, 4, 2, 1)]

        def lanesum(vec):
            for p in perms:
                vec = vec + permute(vec, p)
            return vec

        zf = jnp.zeros((16,), jnp.float32)
        for r in range(rows_w):
            cx, cy, cw, ch = zf, zf, zf, zf
            for k in range(TOPK):
                rv = rows_v[r * TOPK + k, 0:16]
                sel = lane == k
                cx = jnp.where(sel, splat(rv, 0), cx)
                cy = jnp.where(sel, splat(rv, 1), cy)
                cw = jnp.where(sel, splat(rv, 2), cw)
                ch = jnp.where(sel, splat(rv, 3), ch)
            trow = tv[r, 0:16]
            tx = splat(trow, 0)
            ty = splat(trow, 1)
            tw = splat(trow, 2)
            th = splat(trow, 3)

            b1_x1 = tx - tw / 2
            b1_x2 = tx + tw / 2
            b1_y1 = ty - th / 2
            b1_y2 = ty + th / 2
            b2_x1 = cx - cw / 2
            b2_x2 = cx + cw / 2
            b2_y1 = cy - ch / 2
            b2_y2 = cy + ch / 2
            iw = jnp.maximum(jnp.minimum(b1_x2, b2_x2) - jnp.maximum(b1_x1, b2_x1), 0.0)
            ih = jnp.maximum(jnp.minimum(b1_y2, b2_y2) - jnp.maximum(b1_y1, b2_y1), 0.0)
            inter = iw * ih
            w1 = b1_x2 - b1_x1
            h1 = b1_y2 - b1_y1
            w2 = b2_x2 - b2_x1
            h2 = b2_y2 - b2_y1
            union = w1 * h1 + 1e-16 + w2 * h2 - inter
            iou = inter / union
            cw_e = jnp.maximum(b1_x2, b2_x2) - jnp.minimum(b1_x1, b2_x1)
            ch_e = jnp.maximum(b1_y2, b2_y2) - jnp.minimum(b1_y1, b2_y1)
            c_area = cw_e * ch_e + 1e-16
            giou = iou - (c_area - union) / c_area

            g0 = jnp.where(valid, giou, 0.0)
            mu = lanesum(g0) / TOPK
            cdev = jnp.where(valid, giou - mu, 0.0)
            var = lanesum(cdev * cdev) / (TOPK - 1)
            thr = mu + var
            maskv = giou > thr

            ov[r, 0:16] = jnp.where(maskv, cx, 0.0)
            ov[r, 16:32] = jnp.where(maskv, cy, 0.0)
            ov[r, 32:48] = jnp.where(maskv, cw, 0.0)
            ov[r, 48:64] = jnp.where(maskv, ch, 0.0)
            ov[r, 64:80] = giou
            ov[r, 80:96] = jnp.where(maskv, 1.0, 0.0)
            ov[r, 96:112] = zf
            ov[r, 112:128] = zf

        pltpu.sync_copy(ov, out_hbm.at[pl.ds(base, rows_w)])

    return sc_fn


@functools.partial(jax.jit, static_argnames=("interpret",))
def kernel(p_boxes, target, interpret=False):
    n = p_boxes.shape[0]
    nt = target.shape[0]
    npad = ((n + 127) // 128) * 128
    tb = 64

    pc = p_boxes[:, 2:6]
    px = jnp.pad(pc[:, 0], (0, npad - n), constant_values=1e9)
    py = jnp.pad(pc[:, 1], (0, npad - n), constant_values=1e9)
    pref = jnp.stack([px, py], axis=0)
    tref = target[:, 2:6]

    idx9 = pl.pallas_call(
        functools.partial(_topk_body, npad=npad, tb=tb),
        grid=(nt // tb,),
        in_specs=[
            pl.BlockSpec((tb, 4), lambda i: (i, 0)),
            pl.BlockSpec((2, npad), lambda i: (0, 0)),
        ],
        out_specs=pl.BlockSpec((tb, TOPK), lambda i: (i, 0)),
        out_shape=jax.ShapeDtypeStruct((nt, TOPK), jnp.int32),
        compiler_params=pltpu.CompilerParams(
            dimension_semantics=("arbitrary",)),
        interpret=interpret,
    )(tref, pref)

    boxes128 = jnp.pad(pc, ((0, npad - n), (0, 124)))
    tgt128 = jnp.pad(tref, ((0, 0), (0, 124)))
    idx_flat = idx9.reshape(-1)

    sc_fn = _sc_epilogue(nt, npad)
    out = sc_fn(idx_flat, boxes128, tgt128)

    pos = jnp.stack([out[:, 0:TOPK], out[:, 16:16 + TOPK],
                     out[:, 32:32 + TOPK], out[:, 48:48 + TOPK]], axis=-1)
    return pos, out[:, 64:64 + TOPK], out[:, 80:80 + TOPK].astype(bool)


# speedup vs baseline: 2.4744x; 1.0667x over previous
import functools

import jax
import jax.numpy as jnp
from jax import lax
from jax.experimental import pallas as pl
from jax.experimental.pallas import tpu as pltpu
from jax.experimental.pallas import tpu_sc as plsc

TOPK = 9
GRP = 128


def _tc1_body(tref, pref, d_o, gsel_o, *, npad: int, tb: int, gpad: int):
    tx = tref[:, 0:1]
    ty = tref[:, 1:2]
    px = pref[0:1, :]
    py = pref[1:2, :]
    dx = tx - px
    dy = ty - py
    d = jnp.sqrt(dx * dx + dy * dy)
    d_o[...] = d
    ng = npad // GRP
    g = jnp.min(d.reshape(tb, ng, GRP), axis=2)
    pad = jnp.full((tb, gpad - ng), jnp.inf, jnp.float32)
    g = jnp.concatenate([g, pad], axis=1)
    giota = lax.broadcasted_iota(jnp.int32, (tb, gpad), 1)
    cols = []
    for _ in range(TOPK):
        m = jnp.min(g, axis=1, keepdims=True)
        idxv = jnp.min(jnp.where(g == m, giota, gpad), axis=1, keepdims=True)
        g = jnp.where(giota == idxv, jnp.inf, g)
        cols.append(idxv)
    cols += [jnp.zeros((tb, 1), jnp.int32)] * (16 - TOPK)
    gsel_o[...] = jnp.concatenate(cols, axis=1)


def _tc2_body(dref, gselref, idx_o, *, tb: int, npad: int):
    d = dref[...]
    gsel = gselref[...]
    width = 16 * GRP
    lanes = lax.broadcasted_iota(jnp.int32, (tb, width), 1)
    seg = lanes // GRP
    gidx = jnp.take_along_axis(gsel, seg, axis=1) * GRP + lanes % GRP
    d = jnp.where(lanes < TOPK * GRP, d, jnp.inf)
    cols = []
    for _ in range(TOPK):
        m = jnp.min(d, axis=1, keepdims=True)
        wi = jnp.min(jnp.where(d == m, gidx, npad), axis=1, keepdims=True)
        d = jnp.where(gidx == wi, jnp.inf, d)
        cols.append(wi)
    idx_o[...] = jnp.concatenate(cols, axis=1)


def _sc_compact(nt: int, ng: int):
    info = plsc.get_sparse_core_info()
    nc, ns = info.num_cores, info.num_subcores
    nw = nc * ns
    rows_w = nt // nw
    mesh = plsc.VectorSubcoreMesh(core_axis_name="c", subcore_axis_name="s")

    @functools.partial(
        pl.kernel, mesh=mesh,
        out_type=jax.ShapeDtypeStruct((nt * 16, GRP), jnp.float32),
        scratch_types=[
            pltpu.VMEM((rows_w, 16), jnp.int32),
            pltpu.VMEM((rows_w * 16,), jnp.int32),
            pltpu.VMEM((rows_w * 16, GRP), jnp.float32),
            pltpu.SemaphoreType.DMA,
        ],
    )
    def sc_fn(gsel_hbm, dist_hbm, dcomp_hbm, gsel_v, sidx_v, segs_v, sem):
        c = lax.axis_index("c")
        s = lax.axis_index("s")
        wid = s * nc + c
        base = wid * rows_w
        pltpu.sync_copy(gsel_hbm.at[pl.ds(base, rows_w)], gsel_v)

        def row(r, _):
            gv = gsel_v[r]
            sidx_v[pl.ds(r * 16, 16)] = (base + r) * ng + gv
            return 0

        lax.fori_loop(0, rows_w, row, 0, unroll=False)
        pltpu.async_copy(dist_hbm.at[sidx_v], segs_v, sem).wait()
        pltpu.sync_copy(segs_v, dcomp_hbm.at[pl.ds(base * 16, rows_w * 16)])

    return sc_fn


def _sc_epilogue(nt: int, npad: int):
    info = plsc.get_sparse_core_info()
    nc, ns = info.num_cores, info.num_subcores
    nw = nc * ns
    rows_w = nt // nw
    gidx = rows_w * TOPK
    mesh = plsc.VectorSubcoreMesh(core_axis_name="c", subcore_axis_name="s")

    @functools.partial(
        pl.kernel, mesh=mesh,
        out_type=jax.ShapeDtypeStruct((nt, 128), jnp.float32),
        scratch_types=[
            pltpu.VMEM((gidx,), jnp.int32),
            pltpu.VMEM((gidx, 128), jnp.float32),
            pltpu.VMEM((rows_w, 128), jnp.float32),
            pltpu.VMEM((rows_w, 128), jnp.float32),
            pltpu.SemaphoreType.DMA,
        ],
    )
    def sc_fn(idx_hbm, boxes_hbm, tgt_hbm, out_hbm,
              idx_v, rows_v, tv, ov, sem):
        c = lax.axis_index("c")
        s = lax.axis_index("s")
        wid = s * nc + c
        base = wid * rows_w
        pltpu.sync_copy(idx_hbm.at[pl.ds(wid * gidx, gidx)], idx_v)
        pltpu.async_copy(boxes_hbm.at[idx_v], rows_v, sem).wait()
        pltpu.sync_copy(tgt_hbm.at[pl.ds(base, rows_w)], tv)

        lane = lax.iota(jnp.int32, 16)
        valid = lane < TOPK
        comp = [((lane * 0) + cc).reshape(16, 1) for cc in range(4)]
        dnums = lax.GatherDimensionNumbers(
            offset_dims=(), collapsed_slice_dims=(0,), start_index_map=(0,))

        def permute(vec, idx):
            return lax.gather(vec, idx, dnums, (1,),
                              mode=lax.GatherScatterMode.PROMISE_IN_BOUNDS)

        def splat(vec, cc):
            return permute(vec, comp[cc])

        perms = [((lane + sh) % 16).reshape(16, 1) for sh in (8, 4, 2, 1)]

        def lanesum(vec):
            for p in perms:
                vec = vec + permute(vec, p)
            return vec

        zf = jnp.zeros((16,), jnp.float32)
        for r in range(rows_w):
            cx, cy, cw, ch = zf, zf, zf, zf
            for k in range(TOPK):
                rv = rows_v[r * TOPK + k, 0:16]
                sel = lane == k
                cx = jnp.where(sel, splat(rv, 0), cx)
                cy = jnp.where(sel, splat(rv, 1), cy)
                cw = jnp.where(sel, splat(rv, 2), cw)
                ch = jnp.where(sel, splat(rv, 3), ch)
            trow = tv[r, 0:16]
            tx = splat(trow, 0)
            ty = splat(trow, 1)
            tw = splat(trow, 2)
            th = splat(trow, 3)

            b1_x1 = tx - tw / 2
            b1_x2 = tx + tw / 2
            b1_y1 = ty - th / 2
            b1_y2 = ty + th / 2
            b2_x1 = cx - cw / 2
            b2_x2 = cx + cw / 2
            b2_y1 = cy - ch / 2
            b2_y2 = cy + ch / 2
            iw = jnp.maximum(jnp.minimum(b1_x2, b2_x2) - jnp.maximum(b1_x1, b2_x1), 0.0)
            ih = jnp.maximum(jnp.minimum(b1_y2, b2_y2) - jnp.maximum(b1_y1, b2_y1), 0.0)
            inter = iw * ih
            w1 = b1_x2 - b1_x1
            h1 = b1_y2 - b1_y1
            w2 = b2_x2 - b2_x1
            h2 = b2_y2 - b2_y1
            union = w1 * h1 + 1e-16 + w2 * h2 - inter
            iou = inter / union
            cw_e = jnp.maximum(b1_x2, b2_x2) - jnp.minimum(b1_x1, b2_x1)
            ch_e = jnp.maximum(b1_y2, b2_y2) - jnp.minimum(b1_y1, b2_y1)
            c_area = cw_e * ch_e + 1e-16
            giou = iou - (c_area - union) / c_area

            g0 = jnp.where(valid, giou, 0.0)
            mu = lanesum(g0) / TOPK
            cdev = jnp.where(valid, giou - mu, 0.0)
            var = lanesum(cdev * cdev) / (TOPK - 1)
            thr = mu + var
            maskv = giou > thr

            ov[r, 0:16] = jnp.where(maskv, cx, 0.0)
            ov[r, 16:32] = jnp.where(maskv, cy, 0.0)
            ov[r, 32:48] = jnp.where(maskv, cw, 0.0)
            ov[r, 48:64] = jnp.where(maskv, ch, 0.0)
            ov[r, 64:80] = giou
            ov[r, 80:96] = jnp.where(maskv, 1.0, 0.0)
            ov[r, 96:112] = zf
            ov[r, 112:128] = zf

        pltpu.sync_copy(ov, out_hbm.at[pl.ds(base, rows_w)])

    return sc_fn


@functools.partial(jax.jit, static_argnames=("interpret",))
def kernel(p_boxes, target, interpret=False):
    n = p_boxes.shape[0]
    nt = target.shape[0]
    npad = ((n + GRP - 1) // GRP) * GRP
    ng = npad // GRP
    gpad = ((ng + 127) // 128) * 128
    tb = 64

    pc = p_boxes[:, 2:6]
    px = jnp.pad(pc[:, 0], (0, npad - n), constant_values=1e9)
    py = jnp.pad(pc[:, 1], (0, npad - n), constant_values=1e9)
    pref = jnp.stack([px, py], axis=0)
    tref = target[:, 2:6]

    dist, gsel = pl.pallas_call(
        functools.partial(_tc1_body, npad=npad, tb=tb, gpad=gpad),
        grid=(nt // tb,),
        in_specs=[
            pl.BlockSpec((tb, 4), lambda i: (i, 0)),
            pl.BlockSpec((2, npad), lambda i: (0, 0)),
        ],
        out_specs=[pl.BlockSpec((tb, npad), lambda i: (i, 0)),
                   pl.BlockSpec((tb, 16), lambda i: (i, 0))],
        out_shape=[jax.ShapeDtypeStruct((nt, npad), jnp.float32),
                   jax.ShapeDtypeStruct((nt, 16), jnp.int32)],
        compiler_params=pltpu.CompilerParams(
            dimension_semantics=("arbitrary",)),
        interpret=interpret,
    )(tref, pref)

    dist_rows = dist.reshape(nt * ng, GRP)
    dcomp = _sc_compact(nt, ng)(gsel, dist_rows)

    idx9 = pl.pallas_call(
        functools.partial(_tc2_body, tb=tb, npad=npad),
        grid=(nt // tb,),
        in_specs=[
            pl.BlockSpec((tb, 16 * GRP), lambda i: (i, 0)),
            pl.BlockSpec((tb, 16), lambda i: (i, 0)),
        ],
        out_specs=pl.BlockSpec((tb, TOPK), lambda i: (i, 0)),
        out_shape=jax.ShapeDtypeStruct((nt, TOPK), jnp.int32),
        compiler_params=pltpu.CompilerParams(
            dimension_semantics=("arbitrary",)),
        interpret=interpret,
    )(dcomp.reshape(nt, 16 * GRP), gsel)

    boxes128 = jnp.pad(pc, ((0, npad - n), (0, 124)))
    tgt128 = jnp.pad(tref, ((0, 0), (0, 124)))
    idx_flat = idx9.reshape(-1)

    out = _sc_epilogue(nt, npad)(idx_flat, boxes128, tgt128)

    pos = jnp.stack([out[:, 0:TOPK], out[:, 16:16 + TOPK],
                     out[:, 32:32 + TOPK], out[:, 48:48 + TOPK]], axis=-1)
    return pos, out[:, 64:64 + TOPK], out[:, 80:80 + TOPK].astype(bool)
